# baseline (speedup 1.0000x reference)
import functools

import jax
import jax.numpy as jnp
from jax import lax
from jax.experimental import pallas as pl
from jax.experimental.pallas import tpu as pltpu
from jax.experimental.pallas import tpu_sc as plsc

NC = 2
NS = 16
NW = NC * NS
L = 16
CHUNK = 128

MAX_NORM = 1.0
EPS = 1e-7


def _rsqrt(x):
    i = plsc.bitcast(x, jnp.int32)
    i = jnp.int32(0x5F3759DF) - (i >> 1)
    y = plsc.bitcast(i, jnp.float32)
    for _ in range(3):
        y = y * (1.5 - 0.5 * x * y * y)
    return y


def _sqrt(x):
    xs = jnp.maximum(x, 1e-30)
    return xs * _rsqrt(xs)


def _body(p_sub, n_chunks, emb,
          uidx_hbm, iidx_hbm, utab_hbm, itab_hbm, bias_hbm,
          pred_out, ul2_out, il2_out,
          idx_u, idx_i, u_rows, i_rows, bias_v,
          pred_v, ul2_v, il2_v, sem):
    wid = lax.axis_index("s") * NC + lax.axis_index("c")
    base = wid * p_sub

    pltpu.sync_copy(uidx_hbm.at[pl.ds(wid * n_chunks, n_chunks)], idx_u)
    pltpu.sync_copy(iidx_hbm.at[pl.ds(wid * n_chunks, n_chunks)], idx_i)

    copies = []
    for j in range(n_chunks):
        r = pl.ds(j * CHUNK, CHUNK)
        copies.append(pltpu.async_copy(utab_hbm.at[idx_u.at[j]], u_rows.at[r], sem))
        copies.append(pltpu.async_copy(itab_hbm.at[idx_i.at[j]], i_rows.at[r], sem))
        copies.append(pltpu.async_copy(bias_hbm.at[idx_i.at[j]], bias_v.at[r], sem))
    for c in copies:
        c.wait()

    cols = [jnp.full((L,), e, jnp.int32) for e in range(emb)]

    def group(g, _):
        rows = g * L + lax.iota(jnp.int32, L)
        acc_d = jnp.zeros((L,), jnp.float32)
        acc_u2 = jnp.zeros((L,), jnp.float32)
        acc_i2 = jnp.zeros((L,), jnp.float32)
        for e in range(emb):
            u_e = plsc.load_gather(u_rows, [rows, cols[e]])
            i_e = plsc.load_gather(i_rows, [rows, cols[e]])
            acc_d = acc_d + u_e * i_e
            acc_u2 = acc_u2 + u_e * u_e
            acc_i2 = acc_i2 + i_e * i_e
        bias = bias_v[pl.ds(g * L, L)]
        norm_u = _sqrt(acc_u2)
        norm_i = _sqrt(acc_i2)
        su = jnp.minimum(1.0, MAX_NORM / (norm_u + EPS))
        si = jnp.minimum(1.0, MAX_NORM / (norm_i + EPS))
        sl = pl.ds(g * L, L)
        pred_v[sl] = acc_d * (su * si) + bias
        ul2_v[sl] = norm_u * su
        il2_v[sl] = _sqrt(acc_i2 * (si * si) + bias * bias)
        return 0

    lax.fori_loop(0, p_sub // L, group, 0)

    out_sl = pl.ds(base, p_sub)
    pltpu.sync_copy(pred_v, pred_out.at[out_sl])
    pltpu.sync_copy(ul2_v, ul2_out.at[out_sl])
    pltpu.sync_copy(il2_v, il2_out.at[out_sl])


@functools.partial(jax.jit, static_argnums=())
def kernel(user_idx, item_i_idx, user_table, item_table, item_bias_table):
    b = user_idx.shape[0]
    emb = user_table.shape[1]
    p_sub = b // NW
    n_chunks = p_sub // CHUNK
    assert p_sub % L == 0 and p_sub % CHUNK == 0

    uidx2 = user_idx.astype(jnp.int32).reshape(NW * n_chunks, CHUNK)
    iidx2 = item_i_idx.astype(jnp.int32).reshape(NW * n_chunks, CHUNK)
    bias_flat = item_bias_table.reshape(-1)

    mesh = plsc.VectorSubcoreMesh(
        core_axis_name="c", subcore_axis_name="s",
        num_cores=NC, num_subcores=NS)

    f32 = jnp.float32
    out = pl.kernel(
        functools.partial(_body, p_sub, n_chunks, emb),
        out_type=[jax.ShapeDtypeStruct((b,), f32)] * 3,
        mesh=mesh,
        compiler_params=pltpu.CompilerParams(
            needs_layout_passes=False, use_tc_tiling_on_sc=False),
        scratch_types=[
            pltpu.VMEM((n_chunks, CHUNK), jnp.int32),
            pltpu.VMEM((n_chunks, CHUNK), jnp.int32),
            pltpu.VMEM((p_sub, emb), f32),
            pltpu.VMEM((p_sub, emb), f32),
            pltpu.VMEM((p_sub,), f32),
            pltpu.VMEM((p_sub,), f32),
            pltpu.VMEM((p_sub,), f32),
            pltpu.VMEM((p_sub,), f32),
            pltpu.SemaphoreType.DMA,
        ],
    )(uidx2, iidx2, user_table, item_table, bias_flat)
    pred, ul2, il2 = out
    return pred.reshape(b, 1), ul2, il2
